# baseline (speedup 1.0000x reference)
import functools

import jax
import jax.numpy as jnp
import numpy as np
from jax import lax
from jax.experimental import pallas as pl
from jax.experimental.pallas import tpu as pltpu
from jax.experimental.pallas import tpu_sc as plsc

HIDDEN = 128
NUM_RBF = 16
CUTOFF = 5.0
N_NODES = 10000
N_EDGES = 320000

NC = 2
NS = 16
NW = NC * NS
EPW = N_EDGES // NW
CHUNK = 40
NCHUNK = EPW // CHUNK
N_NODES_PAD = 10240
NPT = N_NODES_PAD // NS
ROWBLK = CHUNK
LANES = 16

_LOG2 = float(np.log(2.0))
_PI_OVER_CUTOFF = float(np.pi / CUTOFF)

EDGE_BLK = 3200
NODE_BLK = 2000

_PERM = np.zeros((HIDDEN, HIDDEN), np.float32)
for _i in range(HIDDEN):
    _g, _il = divmod(_i, 32)
    _PERM[_i, _g * 32 + 2 * (_il % 16) + (_il // 16)] = 1.0


def _ssp(v):
    return jnp.logaddexp(v, 0.0) - _LOG2


def _filter_body(ea_ref, dist_ref, fw1_ref, fb1_ref, fw2_ref, fb2_ref, w_ref):
    ea = ea_ref[...].astype(jnp.bfloat16)
    h = _ssp(lax.dot_general(ea, fw1_ref[...].astype(jnp.bfloat16),
                             (((0,), (0,)), ((), ())),
                             preferred_element_type=jnp.float32)
             + fb1_ref[...])
    w = jnp.dot(h.astype(jnp.bfloat16), fw2_ref[...].astype(jnp.bfloat16),
                preferred_element_type=jnp.float32) + fb2_ref[...]
    i = pl.program_id(0)
    rows = jnp.arange(128, dtype=jnp.int32)
    eye = (rows[:, None] == rows[None, :]).astype(jnp.float32)
    ones_col = jnp.ones((128, 1), jnp.float32)
    parts = []
    for r in range(EDGE_BLK // 128):
        d = dist_ref[pl.ds(i * (EDGE_BLK // 128) + r, 1), :]
        cc = 0.5 * (jnp.cos(d * _PI_OVER_CUTOFF) + 1.0)
        cc = cc * (d < CUTOFF).astype(jnp.float32)
        c_col = jnp.dot(eye * cc, ones_col,
                        preferred_element_type=jnp.float32)
        parts.append(w[r * 128:(r + 1) * 128, :] * c_col)
    wc = jnp.concatenate(parts, axis=0)
    w_ref[...] = wc.astype(jnp.bfloat16).reshape(EDGE_BLK // 8, 8, HIDDEN)


def _filter_net(edge_attr, edge_dist, fw1, fb1, fw2, fb2):
    fw2p = fw2 @ _PERM
    fb2p = fb2 @ _PERM
    grid = (N_EDGES // EDGE_BLK,)
    return pl.pallas_call(
        _filter_body,
        grid=grid,
        in_specs=[
            pl.BlockSpec((NUM_RBF, EDGE_BLK), lambda i: (0, i)),
            pl.BlockSpec((N_EDGES // 128, 128), lambda i: (0, 0)),
            pl.BlockSpec((NUM_RBF, HIDDEN), lambda i: (0, 0)),
            pl.BlockSpec((1, HIDDEN), lambda i: (0, 0)),
            pl.BlockSpec((HIDDEN, HIDDEN), lambda i: (0, 0)),
            pl.BlockSpec((1, HIDDEN), lambda i: (0, 0)),
        ],
        out_specs=pl.BlockSpec((EDGE_BLK // 8, 8, HIDDEN), lambda i: (i, 0, 0)),
        out_shape=jax.ShapeDtypeStruct((N_EDGES // 8, 8, HIDDEN),
                                       jnp.bfloat16),
    )(edge_attr.T, edge_dist.reshape(N_EDGES // 128, 128), fw1,
      fb1.reshape(1, HIDDEN), fw2p, fb2p.reshape(1, HIDDEN))


def _atom_body(p_ref, aw1_ref, ab1_ref, aw2_ref, ab2_ref, o_ref):
    xa = p_ref[0] + p_ref[1]
    h = _ssp(jnp.dot(xa.astype(jnp.bfloat16), aw1_ref[...].astype(jnp.bfloat16),
                     preferred_element_type=jnp.float32) + ab1_ref[...])
    o_ref[...] = jnp.dot(h.astype(jnp.bfloat16),
                         aw2_ref[...].astype(jnp.bfloat16),
                         preferred_element_type=jnp.float32) + ab2_ref[...]


def _atom_net(partials, aw1, ab1, aw2, ab2):
    grid = (N_NODES // NODE_BLK,)
    return pl.pallas_call(
        _atom_body,
        grid=grid,
        in_specs=[
            pl.BlockSpec((NC, NODE_BLK, HIDDEN), lambda i: (0, i, 0)),
            pl.BlockSpec((HIDDEN, HIDDEN), lambda i: (0, 0)),
            pl.BlockSpec((1, HIDDEN), lambda i: (0, 0)),
            pl.BlockSpec((HIDDEN, HIDDEN), lambda i: (0, 0)),
            pl.BlockSpec((1, HIDDEN), lambda i: (0, 0)),
        ],
        out_specs=pl.BlockSpec((NODE_BLK, HIDDEN), lambda i: (i, 0)),
        out_shape=jax.ShapeDtypeStruct((N_NODES, HIDDEN), jnp.float32),
    )(partials, aw1, ab1.reshape(1, HIDDEN), aw2, ab2.reshape(1, HIDDEN))


def _sc_body(x_hbm, col_hbm, row_hbm, w_hbm, out_hbm, *sc):
    colb = sc[0:4]
    rowb = sc[4:8]
    wb = sc[8:10]
    xb = sc[10:12]
    mb = sc[12:14]
    acc = sc[14]
    colsem = sc[15:19]
    rowsem = sc[19:23]
    wsem = sc[23:25]
    gsem = sc[25:27]
    ssem = sc[27:29]

    c = lax.axis_index("c")
    s = lax.axis_index("s")
    wid = c * NS + s

    def _zrow(i, carry):
        for l in range(HIDDEN // LANES):
            mb[0][i, pl.ds(l * LANES, LANES)] = jnp.zeros((LANES,),
                                                          jnp.float32)
        return carry
    lax.fori_loop(0, ROWBLK, _zrow, 0)
    for k in range(NPT // ROWBLK):
        pltpu.sync_copy(mb[0], acc.at[pl.ds(s * NPT + k * ROWBLK, ROWBLK)])
    plsc.subcore_barrier()

    ebase = wid * EPW

    def _eoff(j):
        return ebase + j * CHUNK

    def _start_idx(j, q):
        pltpu.async_copy(col_hbm.at[pl.ds(_eoff(j), CHUNK)], colb[q],
                         colsem[q])
        pltpu.async_copy(row_hbm.at[pl.ds(_eoff(j), CHUNK)], rowb[q],
                         rowsem[q])

    def _wait_idx(j, q):
        pltpu.make_async_copy(col_hbm.at[pl.ds(_eoff(j), CHUNK)], colb[q],
                              colsem[q]).wait()
        pltpu.make_async_copy(row_hbm.at[pl.ds(_eoff(j), CHUNK)], rowb[q],
                              rowsem[q]).wait()

    def _start_w(j, b):
        pltpu.async_copy(w_hbm.at[pl.ds(_eoff(j) // 8, CHUNK // 8)], wb[b],
                         wsem[b])

    def _wait_w(j, b):
        pltpu.make_async_copy(w_hbm.at[pl.ds(_eoff(j) // 8, CHUNK // 8)],
                              wb[b], wsem[b]).wait()

    def _start_gather(q, b):
        pltpu.async_copy(x_hbm.at[colb[q]], xb[b], gsem[b])

    def _wait_gather(q, b):
        pltpu.make_async_copy(x_hbm.at[colb[q]], xb[b], gsem[b]).wait()

    def _wait_scatter(b):
        pltpu.make_async_copy(mb[b], acc.at[rowb[0]], ssem[b]).wait()

    _HIMASK = jnp.full((LANES,), -65536, jnp.int32)
    _SH16 = jnp.full((LANES,), 16, jnp.int32)

    def _compute(b):
        def _mgrp(gi, cc):
            for r in range(8):
                rr = r // 2
                base = (r % 2) * 64
                i = gi * 8 + r
                for g in range(HIDDEN // 32):
                    wi = wb[b][gi, rr, pl.ds(base + g * LANES, LANES)]
                    flo = lax.bitcast_convert_type(
                        lax.shift_left(wi, _SH16), jnp.float32)
                    fhi = lax.bitcast_convert_type(
                        lax.bitwise_and(wi, _HIMASK), jnp.float32)
                    slo = pl.ds(g * 32, LANES)
                    shi = pl.ds(g * 32 + LANES, LANES)
                    mb[b][i, slo] = flo * xb[b][i, slo]
                    mb[b][i, shi] = fhi * xb[b][i, shi]
            return cc
        lax.fori_loop(0, CHUNK // 8, _mgrp, 0)

    def _slot(j, b, q, guard_scatter, next_idx, next_gather):
        _wait_w(j, b)
        _wait_gather(q, b)
        if guard_scatter:
            @pl.when(j >= 2)
            def _():
                _wait_scatter(b)
        else:
            _wait_scatter(b)
        _compute(b)
        pltpu.async_copy(mb[b], acc.at[rowb[q]], ssem[b], add=True)
        if next_idx:
            _start_idx(j + 2, (q + 2) % 4)
            _start_w(j + 2, b)
        if next_gather:
            _wait_idx(j + 1, (q + 1) % 4)
            _start_gather((q + 1) % 4, 1 - b)

    _start_idx(0, 0)
    _start_idx(1, 1)
    _wait_idx(0, 0)
    _start_w(0, 0)
    _start_w(1, 1)
    _start_gather(0, 0)

    def _outer(g, carry):
        j = 4 * g
        _slot(j + 0, 0, 0, True, True, True)
        _slot(j + 1, 1, 1, True, True, True)
        _slot(j + 2, 0, 2, True, True, True)
        _slot(j + 3, 1, 3, True, True, True)
        return carry
    lax.fori_loop(0, (NCHUNK - 2) // 4, _outer, 0)
    _slot(NCHUNK - 2, 0, 0, False, False, True)
    _slot(NCHUNK - 1, 1, 1, False, False, False)
    _wait_scatter(0)
    _wait_scatter(1)

    plsc.subcore_barrier()
    for k in range(NPT // ROWBLK):
        sl = pl.ds(s * NPT + k * ROWBLK, ROWBLK)
        pltpu.sync_copy(acc.at[sl], mb[0])
        pltpu.sync_copy(mb[0], out_hbm.at[c, sl])


@functools.partial(
    pl.kernel,
    mesh=plsc.VectorSubcoreMesh(core_axis_name="c", subcore_axis_name="s"),
    out_type=jax.ShapeDtypeStruct((NC, N_NODES_PAD, HIDDEN), jnp.float32),
    scratch_types=(
        [pltpu.VMEM((CHUNK,), jnp.int32)] * 4 +
        [pltpu.VMEM((CHUNK,), jnp.int32)] * 4 +
        [pltpu.VMEM((CHUNK // 8, 4, HIDDEN), jnp.int32)] * 2 +
        [pltpu.VMEM((CHUNK, HIDDEN), jnp.float32)] * 2 +
        [pltpu.VMEM((CHUNK, HIDDEN), jnp.float32)] * 2 +
        [pltpu.VMEM_SHARED((N_NODES_PAD, HIDDEN), jnp.float32)] +
        [pltpu.SemaphoreType.DMA] * 14
    ),
)
def _sc_scatter(x_hbm, col_hbm, row_hbm, w_hbm, out_hbm, *scratch):
    _sc_body(x_hbm, col_hbm, row_hbm, w_hbm, out_hbm, *scratch)


def kernel(x, edge_index, edge_dist, edge_attr,
           fW1, fb1, fW2, fb2, aW1, ab1, aW2, ab2):
    row = edge_index[0].astype(jnp.int32)
    col = edge_index[1].astype(jnp.int32)
    w3d = _filter_net(edge_attr, edge_dist, fW1, fb1, fW2, fb2)
    wi32 = lax.bitcast_convert_type(
        w3d.reshape(N_EDGES // 8, 8, HIDDEN // 2, 2), jnp.int32)
    wi32 = wi32.reshape(N_EDGES // 8, 4, HIDDEN)
    partials = _sc_scatter(x, col, row, wi32)
    return _atom_net(partials, aW1, ab1, aW2, ab2)


# speedup vs baseline: 2.0516x; 2.0516x over previous
import functools

import jax
import jax.numpy as jnp
import numpy as np
from jax import lax
from jax.experimental import pallas as pl
from jax.experimental.pallas import tpu as pltpu
from jax.experimental.pallas import tpu_sc as plsc

HIDDEN = 128
NUM_RBF = 16
CUTOFF = 5.0
N_NODES = 10000
N_EDGES = 320000

NC = 2
NS = 16
NW = NC * NS
EPW = N_EDGES // NW
CHUNK = 40
NCHUNK = EPW // CHUNK
N_NODES_PAD = 10240
NPT = N_NODES_PAD // NS
ROWBLK = CHUNK
LANES = 16

_LOG2 = float(np.log(2.0))
_PI_OVER_CUTOFF = float(np.pi / CUTOFF)

EDGE_BLK = 3200
NODE_BLK = 2000


def _ssp(v):
    return jnp.logaddexp(v, 0.0) - _LOG2


def _filter_body(ea_ref, dist_ref, fw1_ref, fb1_ref, fw2_ref, fb2_ref, w_ref):
    ea = ea_ref[...].astype(jnp.bfloat16)
    h = _ssp(lax.dot_general(ea, fw1_ref[...].astype(jnp.bfloat16),
                             (((0,), (0,)), ((), ())),
                             preferred_element_type=jnp.float32)
             + fb1_ref[...])
    w = jnp.dot(h.astype(jnp.bfloat16), fw2_ref[...].astype(jnp.bfloat16),
                preferred_element_type=jnp.float32) + fb2_ref[...]
    w_ref[...] = w
    i = pl.program_id(0)
    rows = jnp.arange(128, dtype=jnp.int32)
    eye = (rows[:, None] == rows[None, :]).astype(jnp.float32)
    ones_col = jnp.ones((128, 1), jnp.float32)
    for r in range(EDGE_BLK // 128):
        d = dist_ref[pl.ds(i * (EDGE_BLK // 128) + r, 1), :]
        cc = 0.5 * (jnp.cos(d * _PI_OVER_CUTOFF) + 1.0)
        cc = cc * (d < CUTOFF).astype(jnp.float32)
        c_col = jnp.dot(eye * cc, ones_col,
                        preferred_element_type=jnp.float32)
        w_ref[pl.ds(r * 128, 128), :] = w_ref[pl.ds(r * 128, 128), :] * c_col


def _filter_net(edge_attr, edge_dist, fw1, fb1, fw2, fb2):
    grid = (N_EDGES // EDGE_BLK,)
    return pl.pallas_call(
        _filter_body,
        grid=grid,
        in_specs=[
            pl.BlockSpec((NUM_RBF, EDGE_BLK), lambda i: (0, i)),
            pl.BlockSpec((N_EDGES // 128, 128), lambda i: (0, 0)),
            pl.BlockSpec((NUM_RBF, HIDDEN), lambda i: (0, 0)),
            pl.BlockSpec((1, HIDDEN), lambda i: (0, 0)),
            pl.BlockSpec((HIDDEN, HIDDEN), lambda i: (0, 0)),
            pl.BlockSpec((1, HIDDEN), lambda i: (0, 0)),
        ],
        out_specs=pl.BlockSpec((EDGE_BLK, HIDDEN), lambda i: (i, 0)),
        out_shape=jax.ShapeDtypeStruct((N_EDGES, HIDDEN), jnp.float32),
    )(edge_attr.T, edge_dist.reshape(N_EDGES // 128, 128), fw1,
      fb1.reshape(1, HIDDEN), fw2, fb2.reshape(1, HIDDEN))


def _atom_body(p_ref, aw1_ref, ab1_ref, aw2_ref, ab2_ref, o_ref):
    xa = p_ref[0] + p_ref[1]
    h = _ssp(jnp.dot(xa.astype(jnp.bfloat16), aw1_ref[...].astype(jnp.bfloat16),
                     preferred_element_type=jnp.float32) + ab1_ref[...])
    o_ref[...] = jnp.dot(h.astype(jnp.bfloat16),
                         aw2_ref[...].astype(jnp.bfloat16),
                         preferred_element_type=jnp.float32) + ab2_ref[...]


def _atom_net(partials, aw1, ab1, aw2, ab2):
    grid = (N_NODES // NODE_BLK,)
    return pl.pallas_call(
        _atom_body,
        grid=grid,
        in_specs=[
            pl.BlockSpec((NC, NODE_BLK, HIDDEN), lambda i: (0, i, 0)),
            pl.BlockSpec((HIDDEN, HIDDEN), lambda i: (0, 0)),
            pl.BlockSpec((1, HIDDEN), lambda i: (0, 0)),
            pl.BlockSpec((HIDDEN, HIDDEN), lambda i: (0, 0)),
            pl.BlockSpec((1, HIDDEN), lambda i: (0, 0)),
        ],
        out_specs=pl.BlockSpec((NODE_BLK, HIDDEN), lambda i: (i, 0)),
        out_shape=jax.ShapeDtypeStruct((N_NODES, HIDDEN), jnp.float32),
    )(partials, aw1, ab1.reshape(1, HIDDEN), aw2, ab2.reshape(1, HIDDEN))


def _sc_body(x_hbm, col_hbm, row_hbm, w_hbm, out_hbm, *sc):
    colb = sc[0:4]
    rowb = sc[4:8]
    wb = sc[8:10]
    xb = sc[10:12]
    mb = sc[12:14]
    acc = sc[14]
    colsem = sc[15:19]
    rowsem = sc[19:23]
    wsem = sc[23:25]
    gsem = sc[25:27]
    ssem = sc[27:29]

    c = lax.axis_index("c")
    s = lax.axis_index("s")
    wid = c * NS + s

    def _zrow(i, carry):
        for l in range(HIDDEN // LANES):
            mb[0][i, pl.ds(l * LANES, LANES)] = jnp.zeros((LANES,),
                                                          jnp.float32)
        return carry
    lax.fori_loop(0, ROWBLK, _zrow, 0)
    for k in range(NPT // ROWBLK):
        pltpu.sync_copy(mb[0], acc.at[pl.ds(s * NPT + k * ROWBLK, ROWBLK)])
    plsc.subcore_barrier()

    ebase = wid * EPW

    def _eoff(j):
        return ebase + j * CHUNK

    def _start_idx(j, q):
        pltpu.async_copy(col_hbm.at[pl.ds(_eoff(j), CHUNK)], colb[q],
                         colsem[q])
        pltpu.async_copy(row_hbm.at[pl.ds(_eoff(j), CHUNK)], rowb[q],
                         rowsem[q])

    def _wait_idx(j, q):
        pltpu.make_async_copy(col_hbm.at[pl.ds(_eoff(j), CHUNK)], colb[q],
                              colsem[q]).wait()
        pltpu.make_async_copy(row_hbm.at[pl.ds(_eoff(j), CHUNK)], rowb[q],
                              rowsem[q]).wait()

    def _start_w(j, b):
        pltpu.async_copy(w_hbm.at[pl.ds(_eoff(j), CHUNK)], wb[b], wsem[b])

    def _wait_w(j, b):
        pltpu.make_async_copy(w_hbm.at[pl.ds(_eoff(j), CHUNK)], wb[b],
                              wsem[b]).wait()

    def _start_gather(q, b):
        pltpu.async_copy(x_hbm.at[colb[q]], xb[b], gsem[b])

    def _wait_gather(q, b):
        pltpu.make_async_copy(x_hbm.at[colb[q]], xb[b], gsem[b]).wait()

    def _wait_scatter(b):
        pltpu.make_async_copy(mb[b], acc.at[rowb[0]], ssem[b]).wait()

    def _compute(b):
        def _mrow(i, cc):
            for l in range(HIDDEN // LANES):
                sl = pl.ds(l * LANES, LANES)
                mb[b][i, sl] = wb[b][i, sl] * xb[b][i, sl]
            return cc
        lax.fori_loop(0, CHUNK, _mrow, 0)

    def _slot(j, b, q, guard_scatter, next_idx, next_gather):
        _wait_w(j, b)
        _wait_gather(q, b)
        if guard_scatter:
            @pl.when(j >= 2)
            def _():
                _wait_scatter(b)
        else:
            _wait_scatter(b)
        _compute(b)
        pltpu.async_copy(mb[b], acc.at[rowb[q]], ssem[b], add=True)
        if next_idx:
            _start_idx(j + 2, (q + 2) % 4)
            _start_w(j + 2, b)
        if next_gather:
            _wait_idx(j + 1, (q + 1) % 4)
            _start_gather((q + 1) % 4, 1 - b)

    _start_idx(0, 0)
    _start_idx(1, 1)
    _wait_idx(0, 0)
    _start_w(0, 0)
    _start_w(1, 1)
    _start_gather(0, 0)

    def _outer(g, carry):
        j = 4 * g
        _slot(j + 0, 0, 0, True, True, True)
        _slot(j + 1, 1, 1, True, True, True)
        _slot(j + 2, 0, 2, True, True, True)
        _slot(j + 3, 1, 3, True, True, True)
        return carry
    lax.fori_loop(0, (NCHUNK - 2) // 4, _outer, 0)
    _slot(NCHUNK - 2, 0, 0, False, False, True)
    _slot(NCHUNK - 1, 1, 1, False, False, False)
    _wait_scatter(0)
    _wait_scatter(1)

    plsc.subcore_barrier()
    for k in range(NPT // ROWBLK):
        sl = pl.ds(s * NPT + k * ROWBLK, ROWBLK)
        pltpu.sync_copy(acc.at[sl], mb[0])
        pltpu.sync_copy(mb[0], out_hbm.at[c, sl])


@functools.partial(
    pl.kernel,
    mesh=plsc.VectorSubcoreMesh(core_axis_name="c", subcore_axis_name="s"),
    out_type=jax.ShapeDtypeStruct((NC, N_NODES_PAD, HIDDEN), jnp.float32),
    scratch_types=(
        [pltpu.VMEM((CHUNK,), jnp.int32)] * 4 +
        [pltpu.VMEM((CHUNK,), jnp.int32)] * 4 +
        [pltpu.VMEM((CHUNK, HIDDEN), jnp.float32)] * 2 +
        [pltpu.VMEM((CHUNK, HIDDEN), jnp.float32)] * 2 +
        [pltpu.VMEM((CHUNK, HIDDEN), jnp.float32)] * 2 +
        [pltpu.VMEM_SHARED((N_NODES_PAD, HIDDEN), jnp.float32)] +
        [pltpu.SemaphoreType.DMA] * 14
    ),
)
def _sc_scatter(x_hbm, col_hbm, row_hbm, w_hbm, out_hbm, *scratch):
    _sc_body(x_hbm, col_hbm, row_hbm, w_hbm, out_hbm, *scratch)


def kernel(x, edge_index, edge_dist, edge_attr,
           fW1, fb1, fW2, fb2, aW1, ab1, aW2, ab2):
    row = edge_index[0].astype(jnp.int32)
    col = edge_index[1].astype(jnp.int32)
    w = _filter_net(edge_attr, edge_dist, fW1, fb1, fW2, fb2)
    partials = _sc_scatter(x, col, row, w)
    return _atom_net(partials, aW1, ab1, aW2, ab2)
